# Initial kernel scaffold; baseline (speedup 1.0000x reference)
import jax
import jax.numpy as jnp
from jax.experimental import pallas as pl


def kernel(x, pos_embed):
    raise NotImplementedError("write your pallas kernel here")


# speedup vs baseline: 1.4956x; 1.4956x over previous
import jax
import jax.numpy as jnp
from jax.experimental import pallas as pl


S_BLK = 512


def _add_body(x_ref, pe_ref, o_ref):
    o_ref[...] = x_ref[...] + pe_ref[...][None, :, :]


def kernel(x, pos_embed):
    B, S, D = x.shape
    pe = pos_embed[:S]
    grid = (S // S_BLK, B)
    return pl.pallas_call(
        _add_body,
        grid=grid,
        in_specs=[
            pl.BlockSpec((1, S_BLK, D), lambda i, b: (b, i, 0)),
            pl.BlockSpec((S_BLK, D), lambda i, b: (i, 0)),
        ],
        out_specs=pl.BlockSpec((1, S_BLK, D), lambda i, b: (b, i, 0)),
        out_shape=jax.ShapeDtypeStruct((B, S, D), x.dtype),
    )(x, pe)


# speedup vs baseline: 1.6677x; 1.1151x over previous
import jax
import jax.numpy as jnp
from jax.experimental import pallas as pl


S_BLK = 1024


def _add_body(x_ref, pe_ref, o_ref):
    o_ref[...] = x_ref[...] + pe_ref[...][None, :, :]


def kernel(x, pos_embed):
    B, S, D = x.shape
    pe = pos_embed[:S]
    grid = (S // S_BLK, B)
    return pl.pallas_call(
        _add_body,
        grid=grid,
        in_specs=[
            pl.BlockSpec((1, S_BLK, D), lambda i, b: (b, i, 0)),
            pl.BlockSpec((S_BLK, D), lambda i, b: (i, 0)),
        ],
        out_specs=pl.BlockSpec((1, S_BLK, D), lambda i, b: (b, i, 0)),
        out_shape=jax.ShapeDtypeStruct((B, S, D), x.dtype),
    )(x, pe)


# speedup vs baseline: 1.7411x; 1.0440x over previous
import jax
import jax.numpy as jnp
from jax.experimental import pallas as pl


S_BLK = 2048


def _add_body(x_ref, pe_ref, o_ref):
    o_ref[...] = x_ref[...] + pe_ref[...][None, :, :]


def kernel(x, pos_embed):
    B, S, D = x.shape
    pe = pos_embed[:S]
    grid = (S // S_BLK, B)
    return pl.pallas_call(
        _add_body,
        grid=grid,
        in_specs=[
            pl.BlockSpec((1, S_BLK, D), lambda i, b: (b, i, 0)),
            pl.BlockSpec((S_BLK, D), lambda i, b: (i, 0)),
        ],
        out_specs=pl.BlockSpec((1, S_BLK, D), lambda i, b: (b, i, 0)),
        out_shape=jax.ShapeDtypeStruct((B, S, D), x.dtype),
    )(x, pe)
